# baseline (speedup 1.0000x reference)
import functools

import jax
import jax.numpy as jnp
import numpy as np
from jax import lax
from jax.experimental import pallas as pl
from jax.experimental.pallas import tpu as pltpu
from jax.experimental.pallas import tpu_sc as plsc

TOTAL = 2048
SHUFFLE_NUM = 1024
ROWS = 4 * 4096
NUM_WORKERS = 32
ROWS_PER_WORKER = ROWS // NUM_WORKERS
CHUNK_ROWS = 8
CHUNKS = ROWS_PER_WORKER // CHUNK_ROWS
LANES = 16

def _perm_index():
    pkey = jax.random.key(42)
    perm = jax.random.permutation(pkey, TOTAL)
    random_sort = perm[:SHUFFLE_NUM]
    random_index = jnp.sort(random_sort)
    index = jnp.arange(TOTAL, dtype=jnp.int32)
    return index.at[random_index].set(random_sort.astype(jnp.int32))


def _body(x_hbm, idx_hbm, out_hbm, idx_v, in_v, out_v):
    wid = lax.axis_index("c") * 16 + lax.axis_index("s")
    pltpu.sync_copy(idx_hbm, idx_v)
    base = wid * (ROWS_PER_WORKER * TOTAL)

    def chunk_body(c, carry):
        off = base + c * (CHUNK_ROWS * TOTAL)
        pltpu.sync_copy(x_hbm.at[pl.ds(off, CHUNK_ROWS * TOTAL)], in_v)

        def g_body(g, carry2):
            col = g * LANES
            iv = idx_v[pl.ds(col, LANES)]
            for r in range(CHUNK_ROWS):
                vals = plsc.load_gather(in_v, [iv + r * TOTAL])
                out_v[pl.ds(r * TOTAL + col, LANES)] = vals
            return carry2

        lax.fori_loop(0, TOTAL // LANES, g_body, 0, unroll=2)
        pltpu.sync_copy(out_v, out_hbm.at[pl.ds(off, CHUNK_ROWS * TOTAL)])
        return carry

    lax.fori_loop(0, CHUNKS, chunk_body, 0)


@jax.jit
def _shuffle(x_flat, idx):
    mesh = plsc.VectorSubcoreMesh(core_axis_name="c", subcore_axis_name="s")
    k = functools.partial(
        pl.kernel,
        mesh=mesh,
        out_type=jax.ShapeDtypeStruct((ROWS * TOTAL,), jnp.float32),
        scratch_types=[
            pltpu.VMEM((TOTAL,), jnp.int32),
            pltpu.VMEM((CHUNK_ROWS * TOTAL,), jnp.float32),
            pltpu.VMEM((CHUNK_ROWS * TOTAL,), jnp.float32),
        ],
        compiler_params=pltpu.CompilerParams(needs_layout_passes=False),
    )(_body)
    return k(x_flat, idx)


def kernel(input):
    idx = _perm_index()
    out_flat = _shuffle(input.reshape(-1), idx)
    return out_flat.reshape(input.shape)


# speedup vs baseline: 1.9994x; 1.9994x over previous
import functools

import jax
import jax.numpy as jnp
from jax import lax
from jax.experimental import pallas as pl
from jax.experimental.pallas import tpu as pltpu
from jax.experimental.pallas import tpu_sc as plsc

TOTAL = 2048
SHUFFLE_NUM = 1024
ROWS = 4 * 4096
NUM_WORKERS = 32
ROWS_PER_WORKER = ROWS // NUM_WORKERS
CHUNK_ROWS = 8
CHUNKS = ROWS_PER_WORKER // CHUNK_ROWS
CH = CHUNK_ROWS * TOTAL
LANES = 16
GROUPS = TOTAL // LANES


def _perm_index():
    pkey = jax.random.key(42)
    perm = jax.random.permutation(pkey, TOTAL)
    random_sort = perm[:SHUFFLE_NUM]
    random_index = jnp.sort(random_sort)
    index = jnp.arange(TOTAL, dtype=jnp.int32)
    return index.at[random_index].set(random_sort.astype(jnp.int32))


def _body(x_hbm, idx_hbm, out_hbm, idx_v, in0, in1, o0, o1, si0, si1, so0, so1):
    wid = lax.axis_index("c") * 16 + lax.axis_index("s")
    pltpu.sync_copy(idx_hbm, idx_v)
    base = wid * (ROWS_PER_WORKER * TOTAL)
    ins, outs = (in0, in1), (o0, o1)
    isems, osems = (si0, si1), (so0, so1)

    def in_copy(c, b):
        return pltpu.make_async_copy(
            x_hbm.at[pl.ds(base + c * CH, CH)], ins[b], isems[b])

    def out_copy(c, b):
        return pltpu.make_async_copy(
            outs[b], out_hbm.at[pl.ds(base + c * CH, CH)], osems[b])

    def compute(b):
        in_v, out_v = ins[b], outs[b]

        @plsc.parallel_loop(0, GROUPS, unroll=4)
        def _(g):
            col = g * LANES
            iv = idx_v[pl.ds(col, LANES)]
            for r in range(CHUNK_ROWS):
                vals = plsc.load_gather(in_v, [iv + r * TOTAL])
                out_v[pl.ds(r * TOTAL + col, LANES)] = vals

    in_copy(0, 0).start()

    def pair_body(p, carry):
        c0 = p * 2
        for b in range(2):
            c = c0 + b

            @pl.when(c + 1 < CHUNKS)
            def _():
                in_copy(c + 1, 1 - b).start()

            in_copy(c, b).wait()

            @pl.when(c >= 2)
            def _():
                out_copy(c - 2, b).wait()

            compute(b)
            out_copy(c, b).start()
        return carry

    lax.fori_loop(0, CHUNKS // 2, pair_body, 0)
    out_copy(CHUNKS - 2, 0).wait()
    out_copy(CHUNKS - 1, 1).wait()


@jax.jit
def _shuffle(x_flat, idx):
    mesh = plsc.VectorSubcoreMesh(core_axis_name="c", subcore_axis_name="s")
    k = functools.partial(
        pl.kernel,
        mesh=mesh,
        out_type=jax.ShapeDtypeStruct((ROWS * TOTAL,), jnp.float32),
        scratch_types=[
            pltpu.VMEM((TOTAL,), jnp.int32),
            pltpu.VMEM((CH,), jnp.float32),
            pltpu.VMEM((CH,), jnp.float32),
            pltpu.VMEM((CH,), jnp.float32),
            pltpu.VMEM((CH,), jnp.float32),
            pltpu.SemaphoreType.DMA,
            pltpu.SemaphoreType.DMA,
            pltpu.SemaphoreType.DMA,
            pltpu.SemaphoreType.DMA,
        ],
        compiler_params=pltpu.CompilerParams(needs_layout_passes=False),
    )(_body)
    return k(x_flat, idx)


def kernel(input):
    idx = _perm_index()
    out_flat = _shuffle(input.reshape(-1), idx)
    return out_flat.reshape(input.shape)


# speedup vs baseline: 4.6105x; 2.3059x over previous
import functools

import jax
import jax.numpy as jnp
from jax import lax
from jax.experimental import pallas as pl
from jax.experimental.pallas import tpu as pltpu
from jax.experimental.pallas import tpu_sc as plsc

TOTAL = 2048
SHUFFLE_NUM = 1024
ROWS = 4 * 4096
NUM_WORKERS = 32
ROWS_PER_WORKER = ROWS // NUM_WORKERS
CHUNK_ROWS = 8
CHUNKS = ROWS_PER_WORKER // CHUNK_ROWS
LANES = 16
GROUPS = TOTAL // LANES


def _perm_index():
    pkey = jax.random.key(42)
    perm = jax.random.permutation(pkey, TOTAL)
    random_sort = perm[:SHUFFLE_NUM]
    random_index = jnp.sort(random_sort)
    index = jnp.arange(TOTAL, dtype=jnp.int32)
    return index.at[random_index].set(random_sort.astype(jnp.int32))


def _body(x_hbm, idx_hbm, out_hbm, idx_v, in0, in1, o0, o1, si0, si1, so0, so1):
    wid = lax.axis_index("c") * 16 + lax.axis_index("s")
    pltpu.sync_copy(idx_hbm, idx_v)
    row0 = wid * ROWS_PER_WORKER
    ins, outs = (in0, in1), (o0, o1)
    isems, osems = (si0, si1), (so0, so1)

    def in_copy(c, b):
        return pltpu.make_async_copy(
            x_hbm.at[pl.ds(row0 + c * CHUNK_ROWS, CHUNK_ROWS)], ins[b], isems[b])

    def out_copy(c, b):
        return pltpu.make_async_copy(
            outs[b], out_hbm.at[pl.ds(row0 + c * CHUNK_ROWS, CHUNK_ROWS)], osems[b])

    def compute(b):
        in_v, out_v = ins[b], outs[b]

        @plsc.parallel_loop(0, GROUPS, unroll=4)
        def _(g):
            col = g * LANES
            iv = idx_v[pl.ds(col, LANES)]
            for r in range(CHUNK_ROWS):
                rv = jnp.full((LANES,), r, jnp.int32)
                vals = plsc.load_gather(in_v, [rv, iv])
                out_v[r, pl.ds(col, LANES)] = vals

    in_copy(0, 0).start()

    def pair_body(p, carry):
        c0 = p * 2
        for b in range(2):
            c = c0 + b

            @pl.when(c + 1 < CHUNKS)
            def _():
                in_copy(c + 1, 1 - b).start()

            in_copy(c, b).wait()

            @pl.when(c >= 2)
            def _():
                out_copy(c - 2, b).wait()

            compute(b)
            out_copy(c, b).start()
        return carry

    lax.fori_loop(0, CHUNKS // 2, pair_body, 0)
    out_copy(CHUNKS - 2, 0).wait()
    out_copy(CHUNKS - 1, 1).wait()


@jax.jit
def _shuffle(x2, idx):
    mesh = plsc.VectorSubcoreMesh(core_axis_name="c", subcore_axis_name="s")
    k = functools.partial(
        pl.kernel,
        mesh=mesh,
        out_type=jax.ShapeDtypeStruct((ROWS, TOTAL), jnp.float32),
        scratch_types=[
            pltpu.VMEM((TOTAL,), jnp.int32),
            pltpu.VMEM((CHUNK_ROWS, TOTAL), jnp.float32),
            pltpu.VMEM((CHUNK_ROWS, TOTAL), jnp.float32),
            pltpu.VMEM((CHUNK_ROWS, TOTAL), jnp.float32),
            pltpu.VMEM((CHUNK_ROWS, TOTAL), jnp.float32),
            pltpu.SemaphoreType.DMA,
            pltpu.SemaphoreType.DMA,
            pltpu.SemaphoreType.DMA,
            pltpu.SemaphoreType.DMA,
        ],
        compiler_params=pltpu.CompilerParams(needs_layout_passes=False),
    )(_body)
    return k(x2, idx)


def kernel(input):
    idx = _perm_index()
    out2 = _shuffle(input.reshape(ROWS, TOTAL), idx)
    return out2.reshape(input.shape)


# speedup vs baseline: 5.8951x; 1.2786x over previous
import functools

import jax
import jax.numpy as jnp
import numpy as np
from jax import lax
from jax.experimental import pallas as pl
from jax.experimental.pallas import tpu as pltpu
from jax.experimental.pallas import tpu_sc as plsc

TOTAL = 2048
SHUFFLE_NUM = 1024
ROWS = 4 * 4096
NUM_WORKERS = 32
ROWS_PER_WORKER = ROWS // NUM_WORKERS
CHUNK_ROWS = 8
CHUNKS = ROWS_PER_WORKER // CHUNK_ROWS
LANES = 16
GROUPS = TOTAL // LANES


def _perm_index():
    with jax.default_device(jax.local_devices(backend="cpu")[0]):
        pkey = jax.random.key(42)
        perm = jax.random.permutation(pkey, TOTAL)
        random_sort = perm[:SHUFFLE_NUM]
        random_index = jnp.sort(random_sort)
        index = jnp.arange(TOTAL, dtype=jnp.int32)
        index = index.at[random_index].set(random_sort.astype(jnp.int32))
        return np.asarray(index)


_IDX_NP = _perm_index()


def _body(x_hbm, idx_hbm, out_hbm, idx_v, in0, in1, o0, o1, si0, si1, so0, so1):
    wid = lax.axis_index("c") * 16 + lax.axis_index("s")
    pltpu.sync_copy(idx_hbm, idx_v)
    row0 = wid * ROWS_PER_WORKER
    ins, outs = (in0, in1), (o0, o1)
    isems, osems = (si0, si1), (so0, so1)

    def in_copy(c, b):
        return pltpu.make_async_copy(
            x_hbm.at[pl.ds(row0 + c * CHUNK_ROWS, CHUNK_ROWS)], ins[b], isems[b])

    def out_copy(c, b):
        return pltpu.make_async_copy(
            outs[b], out_hbm.at[pl.ds(row0 + c * CHUNK_ROWS, CHUNK_ROWS)], osems[b])

    def compute(b):
        in_v, out_v = ins[b], outs[b]

        @plsc.parallel_loop(0, GROUPS, unroll=8)
        def _(g):
            col = g * LANES
            iv = idx_v[pl.ds(col, LANES)]
            for r in range(CHUNK_ROWS):
                rv = jnp.full((LANES,), r, jnp.int32)
                vals = plsc.load_gather(in_v, [rv, iv])
                out_v[r, pl.ds(col, LANES)] = vals

    in_copy(0, 0).start()

    def pair_body(p, carry):
        c0 = p * 2
        for b in range(2):
            c = c0 + b

            @pl.when(c + 1 < CHUNKS)
            def _():
                in_copy(c + 1, 1 - b).start()

            in_copy(c, b).wait()

            @pl.when(c >= 2)
            def _():
                out_copy(c - 2, b).wait()

            compute(b)
            out_copy(c, b).start()
        return carry

    lax.fori_loop(0, CHUNKS // 2, pair_body, 0)
    out_copy(CHUNKS - 2, 0).wait()
    out_copy(CHUNKS - 1, 1).wait()


@jax.jit
def _shuffle(x2, idx):
    mesh = plsc.VectorSubcoreMesh(core_axis_name="c", subcore_axis_name="s")
    k = functools.partial(
        pl.kernel,
        mesh=mesh,
        out_type=jax.ShapeDtypeStruct((ROWS, TOTAL), jnp.float32),
        scratch_types=[
            pltpu.VMEM((TOTAL,), jnp.int32),
            pltpu.VMEM((CHUNK_ROWS, TOTAL), jnp.float32),
            pltpu.VMEM((CHUNK_ROWS, TOTAL), jnp.float32),
            pltpu.VMEM((CHUNK_ROWS, TOTAL), jnp.float32),
            pltpu.VMEM((CHUNK_ROWS, TOTAL), jnp.float32),
            pltpu.SemaphoreType.DMA,
            pltpu.SemaphoreType.DMA,
            pltpu.SemaphoreType.DMA,
            pltpu.SemaphoreType.DMA,
        ],
        compiler_params=pltpu.CompilerParams(needs_layout_passes=False),
    )(_body)
    return k(x2, idx)


def kernel(input):
    idx = jnp.asarray(_IDX_NP)
    out2 = _shuffle(input.reshape(ROWS, TOTAL), idx)
    return out2.reshape(input.shape)


# speedup vs baseline: 6.0925x; 1.0335x over previous
import functools

import jax
import jax.numpy as jnp
import numpy as np
from jax import lax
from jax.experimental import pallas as pl
from jax.experimental.pallas import tpu as pltpu
from jax.experimental.pallas import tpu_sc as plsc

TOTAL = 2048
SHUFFLE_NUM = 1024
ROWS = 4 * 4096
NUM_WORKERS = 32
ROWS_PER_WORKER = ROWS // NUM_WORKERS
CHUNK_ROWS = 4
CHUNKS = ROWS_PER_WORKER // CHUNK_ROWS
NBUF = 4
LANES = 16
GROUPS = TOTAL // LANES


def _perm_index():
    with jax.default_device(jax.local_devices(backend="cpu")[0]):
        pkey = jax.random.key(42)
        perm = jax.random.permutation(pkey, TOTAL)
        random_sort = perm[:SHUFFLE_NUM]
        random_index = jnp.sort(random_sort)
        index = jnp.arange(TOTAL, dtype=jnp.int32)
        index = index.at[random_index].set(random_sort.astype(jnp.int32))
        return np.asarray(index)


_IDX_NP = _perm_index()


def _body(x_hbm, idx_hbm, out_hbm, idx_v, *refs):
    ins = refs[0:NBUF]
    outs = refs[NBUF:2 * NBUF]
    isems = refs[2 * NBUF:3 * NBUF]
    osems = refs[3 * NBUF:4 * NBUF]
    wid = lax.axis_index("c") * 16 + lax.axis_index("s")
    pltpu.sync_copy(idx_hbm, idx_v)
    row0 = wid * ROWS_PER_WORKER

    def in_copy(c, b):
        return pltpu.make_async_copy(
            x_hbm.at[pl.ds(row0 + c * CHUNK_ROWS, CHUNK_ROWS)], ins[b], isems[b])

    def out_copy(c, b):
        return pltpu.make_async_copy(
            outs[b], out_hbm.at[pl.ds(row0 + c * CHUNK_ROWS, CHUNK_ROWS)], osems[b])

    def compute(b):
        in_v, out_v = ins[b], outs[b]

        @plsc.parallel_loop(0, GROUPS, unroll=8)
        def _(g):
            col = g * LANES
            iv = idx_v[pl.ds(col, LANES)]
            for r in range(CHUNK_ROWS):
                rv = jnp.full((LANES,), r, jnp.int32)
                vals = plsc.load_gather(in_v, [rv, iv])
                out_v[r, pl.ds(col, LANES)] = vals

    for c in range(NBUF - 1):
        in_copy(c, c).start()

    def ring_body(p, carry):
        c0 = p * NBUF
        for b in range(NBUF):
            c = c0 + b

            @pl.when(c + NBUF - 1 < CHUNKS)
            def _():
                in_copy(c + NBUF - 1, (b + NBUF - 1) % NBUF).start()

            in_copy(c, b).wait()

            @pl.when(c >= NBUF)
            def _():
                out_copy(c - NBUF, b).wait()

            compute(b)
            out_copy(c, b).start()
        return carry

    lax.fori_loop(0, CHUNKS // NBUF, ring_body, 0)
    for b in range(NBUF):
        out_copy(CHUNKS - NBUF + b, b).wait()


@jax.jit
def _shuffle(x2, idx):
    mesh = plsc.VectorSubcoreMesh(core_axis_name="c", subcore_axis_name="s")
    k = functools.partial(
        pl.kernel,
        mesh=mesh,
        out_type=jax.ShapeDtypeStruct((ROWS, TOTAL), jnp.float32),
        scratch_types=(
            [pltpu.VMEM((TOTAL,), jnp.int32)]
            + [pltpu.VMEM((CHUNK_ROWS, TOTAL), jnp.float32)] * (2 * NBUF)
            + [pltpu.SemaphoreType.DMA] * (2 * NBUF)
        ),
        compiler_params=pltpu.CompilerParams(needs_layout_passes=False),
    )(_body)
    return k(x2, idx)


def kernel(input):
    idx = jnp.asarray(_IDX_NP)
    out2 = _shuffle(input.reshape(ROWS, TOTAL), idx)
    return out2.reshape(input.shape)
